# baseline (speedup 1.0000x reference)
import functools

import jax
import jax.numpy as jnp
from jax import lax
from jax.experimental import pallas as pl
from jax.experimental.pallas import tpu as pltpu
from jax.experimental.pallas import tpu_sc as plsc

N_NODES = 10000
N_EDGES = 320000
D = 128

NC = 2
NS = 16
NW = NC * NS

EDGES_PER_W = N_EDGES // NW
K = 80
C = EDGES_PER_W // K
HALVES = ((0, 32), (32, 32), (64, 32), (96, 29))
CH = 32

RPT = 632
NP = NS * RPT


def _agg_body(compute_deg, x_hbm, src_hbm, dst_hbm, m_out, deg_out,
              acc_sh, deg_sh, src_v, dst_v, rows0, rows1, rows2, rows3,
              ones_v, sem0, sem1, sem2, sem3):
    cid = lax.axis_index("c")
    sid = lax.axis_index("s")
    wid = sid * NC + cid

    @pl.loop(0, K)
    def _(i):
        for j in range(D // 16):
            rows0[i, pl.ds(j * 16, 16)] = jnp.zeros((16,), jnp.float32)

    for t in range(RPT // K):
        pltpu.sync_copy(rows0, acc_sh.at[pl.ds(sid * RPT + t * K, K)])
    if RPT % K:
        pltpu.sync_copy(rows0.at[pl.ds(0, RPT % K)],
                        acc_sh.at[pl.ds(sid * RPT + (RPT // K) * K, RPT % K)])
    if compute_deg:
        for t in range(4):
            pltpu.sync_copy(rows0.at[0], deg_sh.at[pl.ds(sid * RPT + t * 128, 128)])
        pltpu.sync_copy(rows0.at[0, pl.ds(0, 120)],
                        deg_sh.at[pl.ds(sid * RPT + 512, 120)])
        for j in range(K // 16):
            ones_v[pl.ds(j * 16, 16)] = jnp.ones((16,), jnp.float32)

    plsc.subcore_barrier()

    def start(j, buf, sem):
        pltpu.async_copy(x_hbm.at[src_v.at[j]], buf, sem)

    def wait(buf, sem):
        pltpu.make_async_copy(x_hbm.at[src_v.at[0]], buf, sem).wait()

    def scatter(j, buf):
        pltpu.sync_copy(buf, acc_sh.at[dst_v.at[j]], add=True)
        if compute_deg:
            pltpu.sync_copy(ones_v, deg_sh.at[dst_v.at[j]], add=True)

    bufs = ((rows0, sem0), (rows1, sem1), (rows2, sem2), (rows3, sem3))
    nd = len(bufs)
    ahead = nd - 1
    for off, n in HALVES:
        pltpu.sync_copy(src_hbm.at[wid, pl.ds(off, n)], src_v.at[pl.ds(0, n)])
        pltpu.sync_copy(dst_hbm.at[wid, pl.ds(off, n)], dst_v.at[pl.ds(0, n)])

        for p in range(ahead):
            start(p, *bufs[p])

        nr = (n // nd) * nd

        @pl.loop(0, nr, step=nd)
        def _(j):
            for u in range(nd):
                jj = j + u
                buf, sem = bufs[u]
                wait(buf, sem)

                @pl.when(jj + ahead < n)
                def _():
                    start(jj + ahead, *bufs[(u + ahead) % nd])

                scatter(jj, buf)

        for jj in range(nr, n):
            buf, sem = bufs[jj % nd]
            wait(buf, sem)
            scatter(jj, buf)

    plsc.subcore_barrier()

    pltpu.sync_copy(acc_sh.at[pl.ds(sid * RPT, RPT)],
                    m_out.at[cid, pl.ds(sid * RPT, RPT)])
    if compute_deg:
        @pl.when(sid == 0)
        def _():
            pltpu.sync_copy(deg_sh, deg_out.at[cid, 0])


def _make_agg(compute_deg):
    mesh = plsc.VectorSubcoreMesh(core_axis_name="c", subcore_axis_name="s",
                                  num_cores=NC, num_subcores=NS)
    m_type = jax.ShapeDtypeStruct((NC, NP, D), jnp.float32)
    if compute_deg:
        out_type = [m_type, jax.ShapeDtypeStruct((NC, 1, NP), jnp.float32)]
    else:
        out_type = m_type
    scratch = [
        pltpu.VMEM_SHARED((NP, D), jnp.float32),
        pltpu.VMEM_SHARED((NP,), jnp.float32) if compute_deg else None,
        pltpu.VMEM((CH, K), jnp.int32),
        pltpu.VMEM((CH, K), jnp.int32),
        pltpu.VMEM((K, D), jnp.float32),
        pltpu.VMEM((K, D), jnp.float32),
        pltpu.VMEM((K, D), jnp.float32),
        pltpu.VMEM((K, D), jnp.float32),
        pltpu.VMEM((K,), jnp.float32) if compute_deg else None,
        pltpu.SemaphoreType.DMA,
        pltpu.SemaphoreType.DMA,
        pltpu.SemaphoreType.DMA,
        pltpu.SemaphoreType.DMA,
    ]
    scratch = [s for s in scratch if s is not None]

    if compute_deg:
        def body(x, src, dst, m_out, deg_out, acc, deg, sv, dv, r0, r1, r2,
                 r3, ov, s0, s1, s2, s3):
            _agg_body(True, x, src, dst, m_out, deg_out, acc, deg, sv, dv,
                      r0, r1, r2, r3, ov, s0, s1, s2, s3)
    else:
        def body(x, src, dst, m_out, acc, sv, dv, r0, r1, r2, r3,
                 s0, s1, s2, s3):
            _agg_body(False, x, src, dst, m_out, None, acc, None, sv, dv,
                      r0, r1, r2, r3, None, s0, s1, s2, s3)

    return pl.kernel(body, out_type=out_type, mesh=mesh, scratch_types=scratch,
                     name="gcn_agg_deg" if compute_deg else "gcn_agg")


_AGG_CACHE = {}


def _get_agg(compute_deg):
    if compute_deg not in _AGG_CACHE:
        _AGG_CACHE[compute_deg] = _make_agg(compute_deg)
    return _AGG_CACHE[compute_deg]


def _fused_layer_body(act, m_ref, deg_ref, w_ref, b_ref, out_ref):
    msum = m_ref[0] + m_ref[1]
    d = deg_ref[0] + deg_ref[1]
    inv = 1.0 / jnp.maximum(d, 1.0)
    y = jnp.dot(msum, w_ref[...], preferred_element_type=jnp.float32)
    y = y * inv + b_ref[...]
    if act:
        y = jnp.maximum(y, 0.0)
    out_ref[...] = y


def _make_fused_layer(act, rows_blk=1264):
    grid = (NP // rows_blk,)
    return pl.pallas_call(
        functools.partial(_fused_layer_body, act),
        grid=grid,
        in_specs=[
            pl.BlockSpec((NC, rows_blk, D), lambda i: (0, i, 0)),
            pl.BlockSpec((NC, rows_blk, 1), lambda i: (0, i, 0)),
            pl.BlockSpec((D, D), lambda i: (0, 0)),
            pl.BlockSpec((1, D), lambda i: (0, 0)),
        ],
        out_specs=pl.BlockSpec((rows_blk, D), lambda i: (i, 0)),
        out_shape=jax.ShapeDtypeStruct((NP, D), jnp.float32),
        name="gcn_fused_layer",
    )


_fused_relu = _make_fused_layer(True)
_fused_lin = _make_fused_layer(False)


def kernel(features, edge_index, W0, b0, W1, b1, W2, b2):
    src = edge_index[0].astype(jnp.int32).reshape(NW, C, K)
    dst = edge_index[1].astype(jnp.int32).reshape(NW, C, K)

    m0, deg = _get_agg(True)(features, src, dst)
    deg3 = deg[:, 0, :, None]

    W2p = jnp.zeros((D, D), jnp.float32).at[:, :40].set(W2)
    b2p = jnp.zeros((D,), jnp.float32).at[:40].set(b2)

    h1 = _fused_relu(m0, deg3, W0, b0[None, :])
    m1 = _get_agg(False)(h1, src, dst)
    h2 = _fused_relu(m1, deg3, W1, b1[None, :])
    m2 = _get_agg(False)(h2, src, dst)
    out = _fused_lin(m2, deg3, W2p, b2p[None, :])
    return out[:N_NODES, :40]


# speedup vs baseline: 1.0112x; 1.0112x over previous
import functools

import jax
import jax.numpy as jnp
from jax import lax
from jax.experimental import pallas as pl
from jax.experimental.pallas import tpu as pltpu
from jax.experimental.pallas import tpu_sc as plsc

N_NODES = 10000
N_EDGES = 320000
D = 128

NC = 2
NS = 16
NW = NC * NS

EDGES_PER_W = N_EDGES // NW
K = 80
C = EDGES_PER_W // K
HALVES = ((0, 64), (64, 61))
CH = 64

RPT = 632
NP = NS * RPT


def _agg_body(compute_deg, x_hbm, src_hbm, dst_hbm, m_out, deg_out,
              acc_sh, deg_sh, src_v, dst_v, rows0, rows1, rows2,
              ones_v, sem0, sem1, sem2):
    cid = lax.axis_index("c")
    sid = lax.axis_index("s")
    wid = sid * NC + cid

    @pl.loop(0, K)
    def _(i):
        for j in range(D // 16):
            rows0[i, pl.ds(j * 16, 16)] = jnp.zeros((16,), jnp.float32)

    for t in range(RPT // K):
        pltpu.sync_copy(rows0, acc_sh.at[pl.ds(sid * RPT + t * K, K)])
    if RPT % K:
        pltpu.sync_copy(rows0.at[pl.ds(0, RPT % K)],
                        acc_sh.at[pl.ds(sid * RPT + (RPT // K) * K, RPT % K)])
    if compute_deg:
        for t in range(4):
            pltpu.sync_copy(rows0.at[0], deg_sh.at[pl.ds(sid * RPT + t * 128, 128)])
        pltpu.sync_copy(rows0.at[0, pl.ds(0, 120)],
                        deg_sh.at[pl.ds(sid * RPT + 512, 120)])
        for j in range(K // 16):
            ones_v[pl.ds(j * 16, 16)] = jnp.ones((16,), jnp.float32)

    plsc.subcore_barrier()

    def start(j, buf, sem):
        pltpu.async_copy(x_hbm.at[src_v.at[j]], buf, sem)

    def wait(buf, sem):
        pltpu.make_async_copy(x_hbm.at[src_v.at[0]], buf, sem).wait()

    def scatter(j, buf):
        pltpu.sync_copy(buf, acc_sh.at[dst_v.at[j]], add=True)
        if compute_deg:
            pltpu.sync_copy(ones_v, deg_sh.at[dst_v.at[j]], add=True)

    bufs = ((rows0, sem0), (rows1, sem1), (rows2, sem2))
    nd = len(bufs)
    ahead = nd - 1
    for off, n in HALVES:
        pltpu.sync_copy(src_hbm.at[wid, pl.ds(off, n)], src_v.at[pl.ds(0, n)])
        pltpu.sync_copy(dst_hbm.at[wid, pl.ds(off, n)], dst_v.at[pl.ds(0, n)])

        for p in range(ahead):
            start(p, *bufs[p])

        nr = (n // nd) * nd

        @pl.loop(0, nr, step=nd)
        def _(j):
            for u in range(nd):
                jj = j + u
                buf, sem = bufs[u]
                wait(buf, sem)

                @pl.when(jj + ahead < n)
                def _():
                    start(jj + ahead, *bufs[(u + ahead) % nd])

                scatter(jj, buf)

        for jj in range(nr, n):
            buf, sem = bufs[jj % nd]
            wait(buf, sem)
            scatter(jj, buf)

    plsc.subcore_barrier()

    pltpu.sync_copy(acc_sh.at[pl.ds(sid * RPT, RPT)],
                    m_out.at[cid, pl.ds(sid * RPT, RPT)])
    if compute_deg:
        @pl.when(sid == 0)
        def _():
            pltpu.sync_copy(deg_sh, deg_out.at[cid, 0])


def _make_agg(compute_deg):
    mesh = plsc.VectorSubcoreMesh(core_axis_name="c", subcore_axis_name="s",
                                  num_cores=NC, num_subcores=NS)
    m_type = jax.ShapeDtypeStruct((NC, NP, D), jnp.float32)
    if compute_deg:
        out_type = [m_type, jax.ShapeDtypeStruct((NC, 1, NP), jnp.float32)]
    else:
        out_type = m_type
    scratch = [
        pltpu.VMEM_SHARED((NP, D), jnp.float32),
        pltpu.VMEM_SHARED((NP,), jnp.float32) if compute_deg else None,
        pltpu.VMEM((CH, K), jnp.int32),
        pltpu.VMEM((CH, K), jnp.int32),
        pltpu.VMEM((K, D), jnp.float32),
        pltpu.VMEM((K, D), jnp.float32),
        pltpu.VMEM((K, D), jnp.float32),
        pltpu.VMEM((K,), jnp.float32) if compute_deg else None,
        pltpu.SemaphoreType.DMA,
        pltpu.SemaphoreType.DMA,
        pltpu.SemaphoreType.DMA,
    ]
    scratch = [s for s in scratch if s is not None]

    if compute_deg:
        def body(x, src, dst, m_out, deg_out, acc, deg, sv, dv, r0, r1, r2,
                 ov, s0, s1, s2):
            _agg_body(True, x, src, dst, m_out, deg_out, acc, deg, sv, dv,
                      r0, r1, r2, ov, s0, s1, s2)
    else:
        def body(x, src, dst, m_out, acc, sv, dv, r0, r1, r2, s0, s1, s2):
            _agg_body(False, x, src, dst, m_out, None, acc, None, sv, dv,
                      r0, r1, r2, None, s0, s1, s2)

    return pl.kernel(body, out_type=out_type, mesh=mesh, scratch_types=scratch,
                     name="gcn_agg_deg" if compute_deg else "gcn_agg")


_AGG_CACHE = {}


def _get_agg(compute_deg):
    if compute_deg not in _AGG_CACHE:
        _AGG_CACHE[compute_deg] = _make_agg(compute_deg)
    return _AGG_CACHE[compute_deg]


def _fused_layer_body(act, m_ref, deg_ref, w_ref, b_ref, out_ref):
    msum = m_ref[0] + m_ref[1]
    d = deg_ref[0] + deg_ref[1]
    inv = 1.0 / jnp.maximum(d, 1.0)
    y = jnp.dot(msum, w_ref[...], preferred_element_type=jnp.float32)
    y = y * inv + b_ref[...]
    if act:
        y = jnp.maximum(y, 0.0)
    out_ref[...] = y


def _make_fused_layer(act, rows_blk=1264):
    grid = (NP // rows_blk,)
    return pl.pallas_call(
        functools.partial(_fused_layer_body, act),
        grid=grid,
        in_specs=[
            pl.BlockSpec((NC, rows_blk, D), lambda i: (0, i, 0)),
            pl.BlockSpec((NC, rows_blk, 1), lambda i: (0, i, 0)),
            pl.BlockSpec((D, D), lambda i: (0, 0)),
            pl.BlockSpec((1, D), lambda i: (0, 0)),
        ],
        out_specs=pl.BlockSpec((rows_blk, D), lambda i: (i, 0)),
        out_shape=jax.ShapeDtypeStruct((NP, D), jnp.float32),
        name="gcn_fused_layer",
    )


_fused_relu = _make_fused_layer(True)
_fused_lin = _make_fused_layer(False)


def kernel(features, edge_index, W0, b0, W1, b1, W2, b2):
    src = edge_index[0].astype(jnp.int32).reshape(NW, C, K)
    dst = edge_index[1].astype(jnp.int32).reshape(NW, C, K)

    m0, deg = _get_agg(True)(features, src, dst)
    deg3 = deg[:, 0, :, None]

    W2p = jnp.zeros((D, D), jnp.float32).at[:, :40].set(W2)
    b2p = jnp.zeros((D,), jnp.float32).at[:40].set(b2)

    h1 = _fused_relu(m0, deg3, W0, b0[None, :])
    m1 = _get_agg(False)(h1, src, dst)
    h2 = _fused_relu(m1, deg3, W1, b1[None, :])
    m2 = _get_agg(False)(h2, src, dst)
    out = _fused_lin(m2, deg3, W2p, b2p[None, :])
    return out[:N_NODES, :40]


# speedup vs baseline: 1.0180x; 1.0067x over previous
import functools

import jax
import jax.numpy as jnp
from jax import lax
from jax.experimental import pallas as pl
from jax.experimental.pallas import tpu as pltpu
from jax.experimental.pallas import tpu_sc as plsc

N_NODES = 10000
N_EDGES = 320000
D = 128

NC = 2
NS = 16
NW = NC * NS

EDGES_PER_W = N_EDGES // NW
K = 80
C = EDGES_PER_W // K
HALVES = ((0, 64), (64, 61))
CH = 64

RPT = 632
NP = NS * RPT


def _agg_body(compute_deg, x_hbm, src_hbm, dst_hbm, m_out, deg_out,
              acc_sh, deg_sh, src_v, dst_v, rows0, rows1, rows2,
              ones_v, sem0, sem1, sem2):
    cid = lax.axis_index("c")
    sid = lax.axis_index("s")
    wid = sid * NC + cid

    @pl.loop(0, K)
    def _(i):
        for j in range(D // 16):
            rows0[i, pl.ds(j * 16, 16)] = jnp.zeros((16,), jnp.float32)

    for t in range(RPT // K):
        pltpu.sync_copy(rows0, acc_sh.at[pl.ds(sid * RPT + t * K, K)])
    if RPT % K:
        pltpu.sync_copy(rows0.at[pl.ds(0, RPT % K)],
                        acc_sh.at[pl.ds(sid * RPT + (RPT // K) * K, RPT % K)])
    if compute_deg:
        for t in range(4):
            pltpu.sync_copy(rows0.at[0], deg_sh.at[pl.ds(sid * RPT + t * 128, 128)])
        pltpu.sync_copy(rows0.at[0, pl.ds(0, 120)],
                        deg_sh.at[pl.ds(sid * RPT + 512, 120)])
        for j in range(K // 16):
            ones_v[pl.ds(j * 16, 16)] = jnp.ones((16,), jnp.float32)

    plsc.subcore_barrier()

    def start(j, buf, sem):
        pltpu.async_copy(x_hbm.at[src_v.at[j]], buf, sem)

    def wait(buf, sem):
        pltpu.make_async_copy(x_hbm.at[src_v.at[0]], buf, sem).wait()

    def scatter(j, buf):
        pltpu.sync_copy(buf, acc_sh.at[dst_v.at[j]], add=True)
        if compute_deg:
            pltpu.sync_copy(ones_v, deg_sh.at[dst_v.at[j]], add=True)

    bufs = ((rows0, sem0), (rows1, sem1), (rows2, sem2))
    nd = len(bufs)
    ahead = nd - 1
    for off, n in HALVES:
        pltpu.sync_copy(src_hbm.at[wid, pl.ds(off, n)], src_v.at[pl.ds(0, n)])
        pltpu.sync_copy(dst_hbm.at[wid, pl.ds(off, n)], dst_v.at[pl.ds(0, n)])

        for p in range(ahead):
            start(p, *bufs[p])

        nr = (n // nd) * nd

        @pl.loop(0, nr, step=nd)
        def _(j):
            for u in range(nd):
                jj = j + u
                buf, sem = bufs[u]
                wait(buf, sem)

                @pl.when(jj + ahead < n)
                def _():
                    start(jj + ahead, *bufs[(u + ahead) % nd])

                scatter(jj, buf)

        for jj in range(nr, n):
            buf, sem = bufs[jj % nd]
            wait(buf, sem)
            scatter(jj, buf)

    plsc.subcore_barrier()

    pltpu.sync_copy(acc_sh.at[pl.ds(sid * RPT, RPT)],
                    m_out.at[cid, pl.ds(sid * RPT, RPT)])
    if compute_deg:
        @pl.when(sid == 0)
        def _():
            pltpu.sync_copy(deg_sh, deg_out.at[cid, 0])


def _make_agg(compute_deg):
    mesh = plsc.VectorSubcoreMesh(core_axis_name="c", subcore_axis_name="s",
                                  num_cores=NC, num_subcores=NS)
    m_type = jax.ShapeDtypeStruct((NC, NP, D), jnp.float32)
    if compute_deg:
        out_type = [m_type, jax.ShapeDtypeStruct((NC, 1, NP), jnp.float32)]
    else:
        out_type = m_type
    scratch = [
        pltpu.VMEM_SHARED((NP, D), jnp.float32),
        pltpu.VMEM_SHARED((NP,), jnp.float32) if compute_deg else None,
        pltpu.VMEM((CH, K), jnp.int32),
        pltpu.VMEM((CH, K), jnp.int32),
        pltpu.VMEM((K, D), jnp.float32),
        pltpu.VMEM((K, D), jnp.float32),
        pltpu.VMEM((K, D), jnp.float32),
        pltpu.VMEM((K,), jnp.float32) if compute_deg else None,
        pltpu.SemaphoreType.DMA,
        pltpu.SemaphoreType.DMA,
        pltpu.SemaphoreType.DMA,
    ]
    scratch = [s for s in scratch if s is not None]

    if compute_deg:
        def body(x, src, dst, m_out, deg_out, acc, deg, sv, dv, r0, r1, r2,
                 ov, s0, s1, s2):
            _agg_body(True, x, src, dst, m_out, deg_out, acc, deg, sv, dv,
                      r0, r1, r2, ov, s0, s1, s2)
    else:
        def body(x, src, dst, m_out, acc, sv, dv, r0, r1, r2, s0, s1, s2):
            _agg_body(False, x, src, dst, m_out, None, acc, None, sv, dv,
                      r0, r1, r2, None, s0, s1, s2)

    return pl.kernel(body, out_type=out_type, mesh=mesh, scratch_types=scratch,
                     name="gcn_agg_deg" if compute_deg else "gcn_agg")


_AGG_CACHE = {}


def _get_agg(compute_deg):
    if compute_deg not in _AGG_CACHE:
        _AGG_CACHE[compute_deg] = _make_agg(compute_deg)
    return _AGG_CACHE[compute_deg]


def _fused_layer_body(act, m_ref, deg_ref, w_ref, b_ref, out_ref):
    msum = m_ref[0] + m_ref[1]
    d = deg_ref[0] + deg_ref[1]
    inv = 1.0 / jnp.maximum(d, 1.0)
    y = jnp.dot(msum, w_ref[...], preferred_element_type=jnp.float32)
    y = y * inv + b_ref[...]
    if act:
        y = jnp.maximum(y, 0.0)
    out_ref[...] = y


def _make_fused_layer(act, rows_blk=NP):
    grid = (NP // rows_blk,)
    return pl.pallas_call(
        functools.partial(_fused_layer_body, act),
        grid=grid,
        in_specs=[
            pl.BlockSpec((NC, rows_blk, D), lambda i: (0, i, 0)),
            pl.BlockSpec((NC, rows_blk, 1), lambda i: (0, i, 0)),
            pl.BlockSpec((D, D), lambda i: (0, 0)),
            pl.BlockSpec((1, D), lambda i: (0, 0)),
        ],
        out_specs=pl.BlockSpec((rows_blk, D), lambda i: (i, 0)),
        out_shape=jax.ShapeDtypeStruct((NP, D), jnp.float32),
        name="gcn_fused_layer",
    )


_fused_relu = _make_fused_layer(True)
_fused_lin = _make_fused_layer(False)


def kernel(features, edge_index, W0, b0, W1, b1, W2, b2):
    src = edge_index[0].astype(jnp.int32).reshape(NW, C, K)
    dst = edge_index[1].astype(jnp.int32).reshape(NW, C, K)

    m0, deg = _get_agg(True)(features, src, dst)
    deg3 = deg[:, 0, :, None]

    W2p = jnp.zeros((D, D), jnp.float32).at[:, :40].set(W2)
    b2p = jnp.zeros((D,), jnp.float32).at[:40].set(b2)

    h1 = _fused_relu(m0, deg3, W0, b0[None, :])
    m1 = _get_agg(False)(h1, src, dst)
    h2 = _fused_relu(m1, deg3, W1, b1[None, :])
    m2 = _get_agg(False)(h2, src, dst)
    out = _fused_lin(m2, deg3, W2p, b2p[None, :])
    return out[:N_NODES, :40]
